# baseline (speedup 1.0000x reference)
import jax
import jax.numpy as jnp
from jax import lax
from jax.experimental import pallas as pl

NUM_EXPERTS = 16
TOP_K = 2
_NEG_INF = float("-inf")
NSPLIT = 4
TB = 2048


def _router_block(*refs):
    x_refs = refs[:NSPLIT]
    w_ref = refs[NSPLIT]
    probs_ref, tkl_ref, tki_ref, mask_ref = refs[NSPLIT + 1:]
    w = w_ref[...]
    logits = jnp.concatenate(
        [lax.dot_general(w, xr[...], (((1,), (1,)), ((), ())),
                         preferred_element_type=jnp.float32)
         for xr in x_refs], axis=1)
    iota = lax.broadcasted_iota(jnp.int32, logits.shape, 0)
    m1 = jnp.max(logits, axis=0, keepdims=True)
    i1 = jnp.min(jnp.where(logits == m1, iota, NUM_EXPERTS),
                 axis=0, keepdims=True)
    sel1 = iota == i1
    masked = jnp.where(sel1, _NEG_INF, logits)
    m2 = jnp.max(masked, axis=0, keepdims=True)
    i2 = jnp.min(jnp.where(masked == m2, iota, NUM_EXPERTS),
                 axis=0, keepdims=True)
    sel2 = iota == i2
    keep = sel1 | sel2
    e = jnp.exp(logits - m1)
    denom = 1.0 + jnp.exp(m2 - m1)
    probs_ref[...] = jnp.where(keep, e / denom, 0.0).T
    tkl_ref[...] = jnp.concatenate([m1, m2], axis=0).T
    tki_ref[...] = jnp.concatenate([i1, i2], axis=0).T
    mask_ref[0] = sel1.astype(jnp.float32).T
    mask_ref[1] = sel2.astype(jnp.float32).T


def kernel(x, W_gate):
    Bsz, Tlen, D = x.shape
    E = W_gate.shape[0]
    nt = Bsz * Tlen
    xf = x.reshape(nt, D)
    tb = TB // NSPLIT
    grid = (nt // TB,)

    def x_map(j):
        return lambda i, j=j: (NSPLIT * i + j, 0)

    probs, tkl, tki, mask = pl.pallas_call(
        _router_block,
        grid=grid,
        in_specs=[pl.BlockSpec((tb, D), x_map(j)) for j in range(NSPLIT)]
        + [pl.BlockSpec((E, D), lambda i: (0, 0))],
        out_specs=[
            pl.BlockSpec((TB, E), lambda i: (i, 0)),
            pl.BlockSpec((TB, TOP_K), lambda i: (i, 0)),
            pl.BlockSpec((TB, TOP_K), lambda i: (i, 0)),
            pl.BlockSpec((TOP_K, TB, E), lambda i: (0, i, 0)),
        ],
        out_shape=[
            jax.ShapeDtypeStruct((nt, E), jnp.float32),
            jax.ShapeDtypeStruct((nt, TOP_K), jnp.float32),
            jax.ShapeDtypeStruct((nt, TOP_K), jnp.int32),
            jax.ShapeDtypeStruct((TOP_K, nt, E), jnp.float32),
        ],
    )(*([xf] * NSPLIT + [W_gate]))
    return (probs.reshape(Bsz, Tlen, E),
            tkl.reshape(Bsz, Tlen, TOP_K),
            tki.reshape(Bsz, Tlen, TOP_K),
            mask)


# speedup vs baseline: 1.0061x; 1.0061x over previous
import jax
import jax.numpy as jnp
from jax import lax
from jax.experimental import pallas as pl
from jax.experimental.pallas import tpu as pltpu

NUM_EXPERTS = 16
TOP_K = 2
_NEG_INF = float("-inf")
NBUF = 4
TB = 512


def _router_block(x_hbm, w_ref, probs_ref, tkl_ref, tki_ref, mask_ref,
                  xbuf, sem):
    i = pl.program_id(0)
    nsteps = pl.num_programs(0)

    def issue(block_idx, slot):
        pltpu.make_async_copy(
            x_hbm.at[pl.ds(block_idx * TB, TB), :],
            xbuf.at[slot], sem.at[slot]).start()

    @pl.when(i == 0)
    def _prime():
        for j in range(NBUF - 1):
            issue(j, j)

    nxt = i + NBUF - 1

    @pl.when(nxt < nsteps)
    def _prefetch():
        issue(nxt, nxt % NBUF)

    slot = lax.rem(i, NBUF)
    pltpu.make_async_copy(
        x_hbm.at[pl.ds(0, TB), :], xbuf.at[slot], sem.at[slot]).wait()

    xb = xbuf[slot]
    w = w_ref[...]
    logits = lax.dot_general(w, xb, (((1,), (1,)), ((), ())),
                             preferred_element_type=jnp.float32)
    iota = lax.broadcasted_iota(jnp.int32, logits.shape, 0)
    m1 = jnp.max(logits, axis=0, keepdims=True)
    i1 = jnp.min(jnp.where(logits == m1, iota, NUM_EXPERTS),
                 axis=0, keepdims=True)
    sel1 = iota == i1
    masked = jnp.where(sel1, _NEG_INF, logits)
    m2 = jnp.max(masked, axis=0, keepdims=True)
    i2 = jnp.min(jnp.where(masked == m2, iota, NUM_EXPERTS),
                 axis=0, keepdims=True)
    sel2 = iota == i2
    keep = sel1 | sel2
    e = jnp.exp(logits - m1)
    denom = 1.0 + jnp.exp(m2 - m1)
    probs_ref[...] = jnp.where(keep, e / denom, 0.0).T
    tkl_ref[...] = jnp.concatenate([m1, m2], axis=0).T
    tki_ref[...] = jnp.concatenate([i1, i2], axis=0).T
    mask_ref[0] = sel1.astype(jnp.float32).T
    mask_ref[1] = sel2.astype(jnp.float32).T


def kernel(x, W_gate):
    Bsz, Tlen, D = x.shape
    E = W_gate.shape[0]
    nt = Bsz * Tlen
    xf = x.reshape(nt, D)
    grid = (nt // TB,)
    probs, tkl, tki, mask = pl.pallas_call(
        _router_block,
        grid=grid,
        in_specs=[
            pl.BlockSpec(memory_space=pl.ANY),
            pl.BlockSpec((E, D), lambda i: (0, 0)),
        ],
        out_specs=[
            pl.BlockSpec((TB, E), lambda i: (i, 0)),
            pl.BlockSpec((TB, TOP_K), lambda i: (i, 0)),
            pl.BlockSpec((TB, TOP_K), lambda i: (i, 0)),
            pl.BlockSpec((TOP_K, TB, E), lambda i: (0, i, 0)),
        ],
        out_shape=[
            jax.ShapeDtypeStruct((nt, E), jnp.float32),
            jax.ShapeDtypeStruct((nt, TOP_K), jnp.float32),
            jax.ShapeDtypeStruct((nt, TOP_K), jnp.int32),
            jax.ShapeDtypeStruct((TOP_K, nt, E), jnp.float32),
        ],
        scratch_shapes=[
            pltpu.VMEM((NBUF, TB, D), jnp.float32),
            pltpu.SemaphoreType.DMA((NBUF,)),
        ],
    )(xf, W_gate)
    return (probs.reshape(Bsz, Tlen, E),
            tkl.reshape(Bsz, Tlen, TOP_K),
            tki.reshape(Bsz, Tlen, TOP_K),
            mask)
